# Initial kernel scaffold; baseline (speedup 1.0000x reference)
import jax
import jax.numpy as jnp
from jax.experimental import pallas as pl


def kernel(x, w_down, router_w, w_up):
    raise NotImplementedError("write your pallas kernel here")


# speedup vs baseline: 54.9365x; 54.9365x over previous
import functools

import jax
import jax.numpy as jnp
from jax.experimental import pallas as pl
from jax.experimental.pallas import tpu as pltpu


def _down_body(x_ref, wd_ref, rmat_ref, feats_ref, logits_ref, *, ne, dl):
    feats = jax.lax.dot_general(
        x_ref[...], wd_ref[...],
        dimension_numbers=(((1,), (1,)), ((), ())),
        preferred_element_type=jnp.float32,
    )
    feats_ref[...] = feats
    logits_ref[...] = jnp.dot(feats, rmat_ref[...],
                              preferred_element_type=jnp.float32)


def _up_body(feats_ref, logits_ref, wu_ref, out_ref, aux_ref,
             ps_ref, ls_ref, *, ne, dl, n_tok, d_model, bn):
    i = pl.program_id(0)
    logits = logits_ref[...]
    m = jnp.max(logits, axis=1, keepdims=True)
    ez = jnp.exp(logits - m)
    probs = ez / jnp.sum(ez, axis=1, keepdims=True)

    iota = jax.lax.broadcasted_iota(jnp.int32, (bn, ne), 1)
    m1 = jnp.max(probs, axis=1, keepdims=True)
    i1 = jnp.min(jnp.where(probs == m1, iota, ne), axis=1, keepdims=True)
    mask1 = iota == i1
    pmasked = jnp.where(mask1, -jnp.inf, probs)
    m2 = jnp.max(pmasked, axis=1, keepdims=True)
    i2 = jnp.min(jnp.where(pmasked == m2, iota, ne), axis=1, keepdims=True)
    mask2 = iota == i2
    tot = m1 + m2
    gates = (jnp.where(mask1, m1 / tot, 0.0)
             + jnp.where(mask2, m2 / tot, 0.0))

    @pl.when(i == 0)
    def _init():
        ps_ref[...] = jnp.zeros_like(ps_ref)
        ls_ref[...] = jnp.zeros_like(ls_ref)
        aux_ref[...] = jnp.zeros_like(aux_ref)

    ps_ref[...] += jnp.sum(probs, axis=0, keepdims=True)
    load = mask1.astype(jnp.float32) + mask2.astype(jnp.float32)
    ls_ref[...] += jnp.sum(load, axis=0, keepdims=True)

    acc = jnp.zeros((bn, d_model), jnp.float32)
    for e in range(ne):
        fe = feats_ref[:, e * dl:(e + 1) * dl]
        act = 0.5 * fe * (1.0 + jax.lax.erf(fe * (2.0 ** -0.5)))
        ge = act * gates[:, e:e + 1]
        acc = acc + jnp.dot(ge, wu_ref[e * dl:(e + 1) * dl, :],
                            preferred_element_type=jnp.float32)
    out_ref[...] = acc

    @pl.when(i == pl.num_programs(0) - 1)
    def _fin():
        mp = ps_ref[...] / n_tok
        ml = ls_ref[...] / n_tok
        aux_ref[...] = ne * jnp.sum(mp * ml, axis=(0, 1), keepdims=True)


def kernel(x, w_down, router_w, w_up):
    bsz, t, d_model = x.shape
    ne, dl, _ = w_up.shape
    n_tok = bsz * t
    bn = min(256, n_tok)

    x_flat = x.reshape(n_tok, d_model)
    wu_flat = w_up.reshape(ne * dl, d_model)
    eye = jnp.eye(ne, dtype=jnp.float32)
    rmat = (eye[:, None, :] * router_w[0][None, :, None]).reshape(ne * dl, ne)

    feats, logits = pl.pallas_call(
        functools.partial(_down_body, ne=ne, dl=dl),
        grid=(n_tok // bn,),
        in_specs=[
            pl.BlockSpec((bn, d_model), lambda i: (i, 0)),
            pl.BlockSpec((ne * dl, d_model), lambda i: (0, 0)),
            pl.BlockSpec((ne * dl, ne), lambda i: (0, 0)),
        ],
        out_specs=[
            pl.BlockSpec((bn, ne * dl), lambda i: (i, 0)),
            pl.BlockSpec((bn, ne), lambda i: (i, 0)),
        ],
        out_shape=[
            jax.ShapeDtypeStruct((n_tok, ne * dl), jnp.float32),
            jax.ShapeDtypeStruct((n_tok, ne), jnp.float32),
        ],
    )(x_flat, w_down, rmat)

    out, aux = pl.pallas_call(
        functools.partial(_up_body, ne=ne, dl=dl, n_tok=n_tok,
                          d_model=d_model, bn=bn),
        grid=(n_tok // bn,),
        in_specs=[
            pl.BlockSpec((bn, ne * dl), lambda i: (i, 0)),
            pl.BlockSpec((bn, ne), lambda i: (i, 0)),
            pl.BlockSpec((ne * dl, d_model), lambda i: (0, 0)),
        ],
        out_specs=[
            pl.BlockSpec((bn, d_model), lambda i: (i, 0)),
            pl.BlockSpec((1, 1), lambda i: (0, 0)),
        ],
        out_shape=[
            jax.ShapeDtypeStruct((n_tok, d_model), jnp.float32),
            jax.ShapeDtypeStruct((1, 1), jnp.float32),
        ],
        scratch_shapes=[
            pltpu.VMEM((1, ne), jnp.float32),
            pltpu.VMEM((1, ne), jnp.float32),
        ],
    )(feats, logits, wu_flat)

    return out.reshape(bsz, t, d_model), aux[0, 0]
